# Initial kernel scaffold; baseline (speedup 1.0000x reference)
import jax
import jax.numpy as jnp
from jax.experimental import pallas as pl


def kernel(arm_loc, arm_conf, odm_loc, odm_conf, priors, targets):
    raise NotImplementedError("write your pallas kernel here")


# speedup vs baseline: 35.6451x; 35.6451x over previous
import jax
import jax.numpy as jnp
from jax.experimental import pallas as pl
from jax.experimental.pallas import tpu as pltpu

_NC = 21
_TH = 0.5
_V0, _V1 = 0.1, 0.2
_NPR = 3
_B, _P, _NO = 32, 16320, 12
_R, _C = 8, 2040


def _loss_kernel(t_ref, pri_ref, loc_ref, conf_ref, ll_ref, lc_ref, np_ref):
    b = pl.program_id(0)

    pcx = pri_ref[0]
    pcy = pri_ref[1]
    pw = pri_ref[2]
    ph = pri_ref[3]
    px1 = pcx - pw * 0.5
    py1 = pcy - ph * 0.5
    px2 = pcx + pw * 0.5
    py2 = pcy + ph * 0.5
    area_p = (px2 - px1) * (py2 - py1)

    ridx = jax.lax.broadcasted_iota(jnp.int32, (_R, _C), 0)
    cidx = jax.lax.broadcasted_iota(jnp.int32, (_R, _C), 1)
    pidx = ridx * _C + cidx

    ovs = []
    bt_ov = jnp.full((_R, _C), -1.0, jnp.float32)
    bt_idx = jnp.zeros((_R, _C), jnp.int32)
    for j in range(_NO):
        tx1 = t_ref[0, j, 0]
        ty1 = t_ref[0, j, 1]
        tx2 = t_ref[0, j, 2]
        ty2 = t_ref[0, j, 3]
        iw = jnp.maximum(jnp.minimum(tx2, px2) - jnp.maximum(tx1, px1), 0.0)
        ih = jnp.maximum(jnp.minimum(ty2, py2) - jnp.maximum(ty1, py1), 0.0)
        inter = iw * ih
        at = (tx2 - tx1) * (ty2 - ty1)
        ov = inter / (at + area_p - inter)
        ovs.append(ov)
        upd = ov > bt_ov
        bt_ov = jnp.where(upd, ov, bt_ov)
        bt_idx = jnp.where(upd, j, bt_idx)

    for j in range(_NO):
        ov = ovs[j]
        mj = jnp.max(ov)
        bpi = jnp.min(jnp.where(ov == mj, pidx, _P))
        hit = pidx == bpi
        bt_ov = jnp.where(hit, 2.0, bt_ov)
        bt_idx = jnp.where(hit, j, bt_idx)

    m0 = jnp.zeros((_R, _C), jnp.float32)
    m1 = jnp.zeros((_R, _C), jnp.float32)
    m2 = jnp.zeros((_R, _C), jnp.float32)
    m3 = jnp.zeros((_R, _C), jnp.float32)
    lab = jnp.zeros((_R, _C), jnp.float32)
    for j in range(_NO):
        hit = bt_idx == j
        m0 = jnp.where(hit, t_ref[0, j, 0], m0)
        m1 = jnp.where(hit, t_ref[0, j, 1], m1)
        m2 = jnp.where(hit, t_ref[0, j, 2], m2)
        m3 = jnp.where(hit, t_ref[0, j, 3], m3)
        lab = jnp.where(hit, t_ref[0, j, 4], lab)

    gcx = ((m0 + m2) * 0.5 - pcx) / (_V0 * pw)
    gcy = ((m1 + m3) * 0.5 - pcy) / (_V0 * ph)
    gw = jnp.log((m2 - m0) / pw + 1e-05) / _V1
    gh = jnp.log((m3 - m1) / ph + 1e-05) / _V1

    conf_t = jnp.where(bt_ov < _TH, 0, lab.astype(jnp.int32))
    pos = conf_t > 0
    posf = pos.astype(jnp.float32)

    ll = jnp.float32(0.0)
    for j, lt in enumerate((gcx, gcy, gw, gh)):
        d = loc_ref[0, j] - lt
        ad = jnp.abs(d)
        sl1 = jnp.where(ad < 1.0, 0.5 * d * d, ad - 0.5)
        ll += jnp.sum(sl1 * posf)

    mx = conf_ref[0, 0]
    for c in range(1, _NC):
        mx = jnp.maximum(mx, conf_ref[0, c])
    s = jnp.zeros((_R, _C), jnp.float32)
    for c in range(_NC):
        s += jnp.exp(conf_ref[0, c] - mx)
    lse = jnp.log(s) + mx
    gat = jnp.zeros((_R, _C), jnp.float32)
    for c in range(_NC):
        gat = jnp.where(conf_t == c, conf_ref[0, c], gat)
    ce = lse - gat

    lc_pos = jnp.sum(ce * posf)
    npos = jnp.sum(conf_t > 0, dtype=jnp.int32)
    k = jnp.minimum(_NPR * npos, _P - 1)

    v = jnp.where(pos, 0.0, ce)
    vb = jax.lax.bitcast_convert_type(v, jnp.int32)

    def body(i, acc):
        cand = acc | (jnp.int32(1) << (30 - i))
        cnt = jnp.sum((vb >= cand).astype(jnp.int32))
        return jnp.where(cnt >= k, cand, acc)

    tb = jax.lax.fori_loop(0, 31, body, jnp.int32(0))
    cgt = jnp.sum((vb > tb).astype(jnp.int32))
    sgt = jnp.sum(jnp.where(vb > tb, v, 0.0))
    tf = jax.lax.bitcast_convert_type(tb, jnp.float32)
    lc_neg = jnp.where(k > 0, sgt + (k - cgt).astype(jnp.float32) * tf, 0.0)

    @pl.when(b == 0)
    def _init():
        ll_ref[0, 0] = 0.0
        lc_ref[0, 0] = 0.0
        np_ref[0, 0] = 0

    ll_ref[0, 0] += ll
    lc_ref[0, 0] += lc_pos + lc_neg
    np_ref[0, 0] += npos


def kernel(arm_loc, arm_conf, odm_loc, odm_conf, priors, targets):
    del odm_loc, odm_conf
    pri = priors.T.reshape(4, _R, _C)
    loc = arm_loc.transpose(0, 2, 1).reshape(_B, 4, _R, _C)
    conf = arm_conf.transpose(0, 2, 1).reshape(_B, _NC, _R, _C)
    ll, lc, npos = pl.pallas_call(
        _loss_kernel,
        grid=(_B,),
        in_specs=[
            pl.BlockSpec((1, _NO, 5), lambda b: (b, 0, 0),
                         memory_space=pltpu.SMEM),
            pl.BlockSpec((4, _R, _C), lambda b: (0, 0, 0)),
            pl.BlockSpec((1, 4, _R, _C), lambda b: (b, 0, 0, 0)),
            pl.BlockSpec((1, _NC, _R, _C), lambda b: (b, 0, 0, 0)),
        ],
        out_specs=[
            pl.BlockSpec((1, 1), lambda b: (0, 0), memory_space=pltpu.SMEM),
            pl.BlockSpec((1, 1), lambda b: (0, 0), memory_space=pltpu.SMEM),
            pl.BlockSpec((1, 1), lambda b: (0, 0), memory_space=pltpu.SMEM),
        ],
        out_shape=[
            jax.ShapeDtypeStruct((1, 1), jnp.float32),
            jax.ShapeDtypeStruct((1, 1), jnp.float32),
            jax.ShapeDtypeStruct((1, 1), jnp.int32),
        ],
    )(targets, pri, loc, conf)
    n = npos[0, 0].astype(jnp.float32)
    return ll[0, 0] / n, lc[0, 0] / n


# speedup vs baseline: 54.9129x; 1.5405x over previous
import jax
import jax.numpy as jnp
from jax.experimental import pallas as pl
from jax.experimental.pallas import tpu as pltpu

_NC = 21
_TH = 0.5
_V0, _V1 = 0.1, 0.2
_NPR = 3
_B, _P, _NO = 32, 16320, 12
_R, _C = 8, 2040


def _loss_kernel(t_ref, pri_ref, loc_ref, conf_ref, ll_ref, lc_ref, np_ref,
                 v_scr, p_scr):
    b = pl.program_id(0)

    pcx = pri_ref[0]
    pcy = pri_ref[1]
    pw = pri_ref[2]
    ph = pri_ref[3]
    px1 = pcx - pw * 0.5
    py1 = pcy - ph * 0.5
    px2 = pcx + pw * 0.5
    py2 = pcy + ph * 0.5
    area_p = (px2 - px1) * (py2 - py1)

    ridx = jax.lax.broadcasted_iota(jnp.int32, (_R, _C), 0)
    cidx = jax.lax.broadcasted_iota(jnp.int32, (_R, _C), 1)
    pidx = ridx * _C + cidx

    ovs = []
    bt_ov = jnp.full((_R, _C), -1.0, jnp.float32)
    bt_idx = jnp.zeros((_R, _C), jnp.int32)
    for j in range(_NO):
        tx1 = t_ref[0, j, 0]
        ty1 = t_ref[0, j, 1]
        tx2 = t_ref[0, j, 2]
        ty2 = t_ref[0, j, 3]
        iw = jnp.maximum(jnp.minimum(tx2, px2) - jnp.maximum(tx1, px1), 0.0)
        ih = jnp.maximum(jnp.minimum(ty2, py2) - jnp.maximum(ty1, py1), 0.0)
        inter = iw * ih
        at = (tx2 - tx1) * (ty2 - ty1)
        ov = inter / (at + area_p - inter)
        ovs.append(ov)
        upd = ov > bt_ov
        bt_ov = jnp.where(upd, ov, bt_ov)
        bt_idx = jnp.where(upd, j, bt_idx)

    for j in range(_NO):
        ov = ovs[j]
        mj = jnp.max(ov)
        bpi = jnp.min(jnp.where(ov == mj, pidx, _P))
        hit = pidx == bpi
        bt_ov = jnp.where(hit, 2.0, bt_ov)
        bt_idx = jnp.where(hit, j, bt_idx)

    m0 = jnp.zeros((_R, _C), jnp.float32)
    m1 = jnp.zeros((_R, _C), jnp.float32)
    m2 = jnp.zeros((_R, _C), jnp.float32)
    m3 = jnp.zeros((_R, _C), jnp.float32)
    lab = jnp.zeros((_R, _C), jnp.float32)
    for j in range(_NO):
        hit = bt_idx == j
        m0 = jnp.where(hit, t_ref[0, j, 0], m0)
        m1 = jnp.where(hit, t_ref[0, j, 1], m1)
        m2 = jnp.where(hit, t_ref[0, j, 2], m2)
        m3 = jnp.where(hit, t_ref[0, j, 3], m3)
        lab = jnp.where(hit, t_ref[0, j, 4], lab)

    gcx = ((m0 + m2) * 0.5 - pcx) / (_V0 * pw)
    gcy = ((m1 + m3) * 0.5 - pcy) / (_V0 * ph)
    gw = jnp.log((m2 - m0) / pw + 1e-05) / _V1
    gh = jnp.log((m3 - m1) / ph + 1e-05) / _V1

    conf_t = jnp.where(bt_ov < _TH, 0, lab.astype(jnp.int32))
    pos = conf_t > 0
    posf = pos.astype(jnp.float32)

    ll = jnp.float32(0.0)
    for j, lt in enumerate((gcx, gcy, gw, gh)):
        d = loc_ref[0, j] - lt
        ad = jnp.abs(d)
        sl1 = jnp.where(ad < 1.0, 0.5 * d * d, ad - 0.5)
        ll += jnp.sum(sl1 * posf)

    mx = conf_ref[0, 0]
    for c in range(1, _NC):
        mx = jnp.maximum(mx, conf_ref[0, c])
    s = jnp.zeros((_R, _C), jnp.float32)
    for c in range(_NC):
        s += jnp.exp(conf_ref[0, c] - mx)
    lse = jnp.log(s) + mx
    gat = jnp.zeros((_R, _C), jnp.float32)
    for c in range(_NC):
        gat = jnp.where(conf_t == c, conf_ref[0, c], gat)
    ce = lse - gat

    lc_pos = jnp.sum(ce * posf)

    v_scr[b] = jnp.where(pos, 0.0, ce)
    p_scr[b] = posf

    @pl.when(b == 0)
    def _init():
        ll_ref[0, 0] = 0.0
        lc_ref[0, 0] = 0.0
        np_ref[0, 0] = 0

    ll_ref[0, 0] += ll
    lc_ref[0, 0] += lc_pos

    @pl.when(b == _B - 1)
    def _final():
        v = v_scr[...]
        vb = jax.lax.bitcast_convert_type(v, jnp.int32)
        npos_f = jnp.sum(jnp.sum(p_scr[...], axis=2, keepdims=True),
                         axis=1, keepdims=True)
        npos = npos_f.astype(jnp.int32)
        k = jnp.minimum(_NPR * npos, _P - 1)

        def body(i, acc):
            cand = acc | (jnp.int32(1) << (30 - i))
            hit = (vb >= cand).astype(jnp.int32)
            cnt = jnp.sum(jnp.sum(hit, axis=2, keepdims=True),
                          axis=1, keepdims=True)
            return jnp.where(cnt >= k, cand, acc)

        tb = jax.lax.fori_loop(
            0, 31, body, jnp.zeros((_B, 1, 1), jnp.int32))
        gt = vb > tb
        cgt = jnp.sum(jnp.sum(gt.astype(jnp.int32), axis=2, keepdims=True),
                      axis=1, keepdims=True)
        sgt = jnp.sum(jnp.sum(jnp.where(gt, v, 0.0), axis=2, keepdims=True),
                      axis=1, keepdims=True)
        tf = jax.lax.bitcast_convert_type(tb, jnp.float32)
        lc_neg = jnp.where(k > 0, sgt + (k - cgt).astype(jnp.float32) * tf,
                           0.0)
        lc_ref[0, 0] += jnp.sum(lc_neg)
        np_ref[0, 0] = jnp.sum(npos)


def kernel(arm_loc, arm_conf, odm_loc, odm_conf, priors, targets):
    del odm_loc, odm_conf
    pri = priors.T.reshape(4, _R, _C)
    loc = arm_loc.transpose(0, 2, 1).reshape(_B, 4, _R, _C)
    conf = arm_conf.transpose(0, 2, 1).reshape(_B, _NC, _R, _C)
    ll, lc, npos = pl.pallas_call(
        _loss_kernel,
        grid=(_B,),
        in_specs=[
            pl.BlockSpec((1, _NO, 5), lambda b: (b, 0, 0),
                         memory_space=pltpu.SMEM),
            pl.BlockSpec((4, _R, _C), lambda b: (0, 0, 0)),
            pl.BlockSpec((1, 4, _R, _C), lambda b: (b, 0, 0, 0)),
            pl.BlockSpec((1, _NC, _R, _C), lambda b: (b, 0, 0, 0)),
        ],
        out_specs=[
            pl.BlockSpec((1, 1), lambda b: (0, 0), memory_space=pltpu.SMEM),
            pl.BlockSpec((1, 1), lambda b: (0, 0), memory_space=pltpu.SMEM),
            pl.BlockSpec((1, 1), lambda b: (0, 0), memory_space=pltpu.SMEM),
        ],
        out_shape=[
            jax.ShapeDtypeStruct((1, 1), jnp.float32),
            jax.ShapeDtypeStruct((1, 1), jnp.float32),
            jax.ShapeDtypeStruct((1, 1), jnp.int32),
        ],
        scratch_shapes=[
            pltpu.VMEM((_B, _R, _C), jnp.float32),
            pltpu.VMEM((_B, _R, _C), jnp.float32),
        ],
    )(targets, pri, loc, conf)
    n = npos[0, 0].astype(jnp.float32)
    return ll[0, 0] / n, lc[0, 0] / n


# speedup vs baseline: 78.1098x; 1.4224x over previous
import jax
import jax.numpy as jnp
from jax.experimental import pallas as pl
from jax.experimental.pallas import tpu as pltpu

_NC = 21
_TH = 0.5
_V0, _V1 = 0.1, 0.2
_NPR = 3
_B, _P, _NO = 32, 16320, 12
_R, _C = 8, 2040


def _loss_kernel(t_ref, pri_ref, loc_ref, conf_ref, ll_ref, lc_ref, np_ref,
                 v_scr, p_scr):
    b = pl.program_id(0)

    pcx = pri_ref[0]
    pcy = pri_ref[1]
    pw = pri_ref[2]
    ph = pri_ref[3]
    px1 = pcx - pw * 0.5
    py1 = pcy - ph * 0.5
    px2 = pcx + pw * 0.5
    py2 = pcy + ph * 0.5
    area_p = (px2 - px1) * (py2 - py1)

    ridx = jax.lax.broadcasted_iota(jnp.int32, (_R, _C), 0)
    cidx = jax.lax.broadcasted_iota(jnp.int32, (_R, _C), 1)
    pidx = ridx * _C + cidx

    ovs = []
    bt_ov = jnp.full((_R, _C), -1.0, jnp.float32)
    bt_idx = jnp.zeros((_R, _C), jnp.int32)
    for j in range(_NO):
        tx1 = t_ref[0, j, 0]
        ty1 = t_ref[0, j, 1]
        tx2 = t_ref[0, j, 2]
        ty2 = t_ref[0, j, 3]
        iw = jnp.maximum(jnp.minimum(tx2, px2) - jnp.maximum(tx1, px1), 0.0)
        ih = jnp.maximum(jnp.minimum(ty2, py2) - jnp.maximum(ty1, py1), 0.0)
        inter = iw * ih
        at = (tx2 - tx1) * (ty2 - ty1)
        ov = inter / (at + area_p - inter)
        ovs.append(ov)
        upd = ov > bt_ov
        bt_ov = jnp.where(upd, ov, bt_ov)
        bt_idx = jnp.where(upd, j, bt_idx)

    ov_stk = jnp.stack(ovs)
    mj = jnp.max(jnp.max(ov_stk, axis=2, keepdims=True),
                 axis=1, keepdims=True)
    masked = jnp.where(ov_stk == mj, pidx[None], _P)
    bpi = jnp.min(jnp.min(masked, axis=2, keepdims=True),
                  axis=1, keepdims=True)
    hits = pidx[None] == bpi
    for j in range(_NO):
        bt_ov = jnp.where(hits[j], 2.0, bt_ov)
        bt_idx = jnp.where(hits[j], j, bt_idx)

    m0 = jnp.zeros((_R, _C), jnp.float32)
    m1 = jnp.zeros((_R, _C), jnp.float32)
    m2 = jnp.zeros((_R, _C), jnp.float32)
    m3 = jnp.zeros((_R, _C), jnp.float32)
    lab = jnp.zeros((_R, _C), jnp.float32)
    for j in range(_NO):
        hit = bt_idx == j
        m0 = jnp.where(hit, t_ref[0, j, 0], m0)
        m1 = jnp.where(hit, t_ref[0, j, 1], m1)
        m2 = jnp.where(hit, t_ref[0, j, 2], m2)
        m3 = jnp.where(hit, t_ref[0, j, 3], m3)
        lab = jnp.where(hit, t_ref[0, j, 4], lab)

    gcx = ((m0 + m2) * 0.5 - pcx) / (_V0 * pw)
    gcy = ((m1 + m3) * 0.5 - pcy) / (_V0 * ph)
    gw = jnp.log((m2 - m0) / pw + 1e-05) / _V1
    gh = jnp.log((m3 - m1) / ph + 1e-05) / _V1

    conf_t = jnp.where(bt_ov < _TH, 0, lab.astype(jnp.int32))
    pos = conf_t > 0
    posf = pos.astype(jnp.float32)

    ll = jnp.float32(0.0)
    for j, lt in enumerate((gcx, gcy, gw, gh)):
        d = loc_ref[0, j] - lt
        ad = jnp.abs(d)
        sl1 = jnp.where(ad < 1.0, 0.5 * d * d, ad - 0.5)
        ll += jnp.sum(sl1 * posf)

    mx = conf_ref[0, 0]
    for c in range(1, _NC):
        mx = jnp.maximum(mx, conf_ref[0, c])
    s = jnp.zeros((_R, _C), jnp.float32)
    for c in range(_NC):
        s += jnp.exp(conf_ref[0, c] - mx)
    lse = jnp.log(s) + mx
    gat = jnp.zeros((_R, _C), jnp.float32)
    for c in range(_NC):
        gat = jnp.where(conf_t == c, conf_ref[0, c], gat)
    ce = lse - gat

    v_scr[b] = ce
    p_scr[b] = posf

    @pl.when(b == 0)
    def _init():
        ll_ref[0, 0] = 0.0
        lc_ref[0, 0] = 0.0
        np_ref[0, 0] = 0

    ll_ref[0, 0] += ll

    @pl.when(b == _B - 1)
    def _final():
        def rsum(x):
            return jnp.sum(jnp.sum(x, axis=2, keepdims=True),
                           axis=1, keepdims=True)

        ce_all = v_scr[...]
        pall = p_scr[...]
        posm = pall > 0.0
        lc_pos = jnp.sum(jnp.where(posm, ce_all, 0.0))
        npos = rsum(pall).astype(jnp.int32)
        k = jnp.minimum(_NPR * npos, _P - 1)

        v = jnp.where(posm, 0.0, ce_all)
        vb = jax.lax.bitcast_convert_type(v, jnp.int32)

        def body(i, acc):
            cand = acc | (jnp.int32(1) << (30 - i))
            cnt = rsum((vb >= cand).astype(jnp.int32))
            return jnp.where(cnt >= k, cand, acc)

        tb = jax.lax.fori_loop(
            0, 23, body, jnp.zeros((_B, 1, 1), jnp.int32))
        gt = vb > tb
        cgt = rsum(gt.astype(jnp.int32))
        sgt = rsum(jnp.where(gt, v, 0.0))
        tf = jax.lax.bitcast_convert_type(tb, jnp.float32)
        lc_neg = jnp.where(k > 0, sgt + (k - cgt).astype(jnp.float32) * tf,
                           0.0)
        lc_ref[0, 0] += lc_pos + jnp.sum(lc_neg)
        np_ref[0, 0] = jnp.sum(npos)


def kernel(arm_loc, arm_conf, odm_loc, odm_conf, priors, targets):
    del odm_loc, odm_conf
    pri = priors.T.reshape(4, _R, _C)
    loc = arm_loc.transpose(0, 2, 1).reshape(_B, 4, _R, _C)
    conf = arm_conf.transpose(0, 2, 1).reshape(_B, _NC, _R, _C)
    ll, lc, npos = pl.pallas_call(
        _loss_kernel,
        grid=(_B,),
        in_specs=[
            pl.BlockSpec((1, _NO, 5), lambda b: (b, 0, 0),
                         memory_space=pltpu.SMEM),
            pl.BlockSpec((4, _R, _C), lambda b: (0, 0, 0)),
            pl.BlockSpec((1, 4, _R, _C), lambda b: (b, 0, 0, 0)),
            pl.BlockSpec((1, _NC, _R, _C), lambda b: (b, 0, 0, 0)),
        ],
        out_specs=[
            pl.BlockSpec((1, 1), lambda b: (0, 0), memory_space=pltpu.SMEM),
            pl.BlockSpec((1, 1), lambda b: (0, 0), memory_space=pltpu.SMEM),
            pl.BlockSpec((1, 1), lambda b: (0, 0), memory_space=pltpu.SMEM),
        ],
        out_shape=[
            jax.ShapeDtypeStruct((1, 1), jnp.float32),
            jax.ShapeDtypeStruct((1, 1), jnp.float32),
            jax.ShapeDtypeStruct((1, 1), jnp.int32),
        ],
        scratch_shapes=[
            pltpu.VMEM((_B, _R, _C), jnp.float32),
            pltpu.VMEM((_B, _R, _C), jnp.float32),
        ],
    )(targets, pri, loc, conf)
    n = npos[0, 0].astype(jnp.float32)
    return ll[0, 0] / n, lc[0, 0] / n
